# baseline (speedup 1.0000x reference)
import functools

import jax
import jax.numpy as jnp
from jax import lax
from jax.experimental import pallas as pl
from jax.experimental.pallas import tpu as pltpu
from jax.experimental.pallas import tpu_sc as plsc

EMBED = 32
BATCH = 16384
NC, NS = 2, 16
NW = NC * NS
PW = BATCH // NW
NBUF = 16


@functools.partial(
    pl.kernel,
    mesh=plsc.VectorSubcoreMesh(core_axis_name="c", subcore_axis_name="s"),
    out_type=jax.ShapeDtypeStruct((EMBED, BATCH), jnp.float32),
    scratch_types=[
        pltpu.VMEM((PW + 16,), jnp.int32),
        pltpu.VMEM((NBUF, EMBED, 128), jnp.float32),
        pltpu.VMEM((EMBED, PW), jnp.float32),
        pltpu.SemaphoreType.DMA,
    ],
    compiler_params=pltpu.CompilerParams(
        use_tc_tiling_on_sc=True, needs_layout_passes=False),
)
def _gather_sc(idx_hbm, tab_hbm, out_hbm, idx_v, blk_v, out_v, sem):
    wid = lax.axis_index("s") * NC + lax.axis_index("c")
    base = wid * PW
    pltpu.sync_copy(idx_hbm.at[pl.ds(base, PW)], idx_v.at[pl.ds(0, PW)])

    row_lo = lax.iota(jnp.int32, 16)
    row_hi = row_lo + 16

    def idx_at(k):
        return idx_v[pl.ds(k, 16)][0]

    def fire(k, slot):
        start = pl.multiple_of((idx_at(k) >> 7) * 128, 128)
        pltpu.async_copy(
            tab_hbm.at[:, pl.ds(start, 128)], blk_v.at[slot], sem)

    for b in range(NBUF):
        fire(b, b)

    def body(k, carry):
        slot = lax.rem(k, NBUF)
        pltpu.make_async_copy(
            tab_hbm.at[:, pl.ds(0, 128)], blk_v.at[slot], sem).wait()
        li = jnp.broadcast_to(idx_at(k) & 127, (16,))
        sv = jnp.broadcast_to(slot, (16,))
        col = jnp.broadcast_to(k, (16,))
        lo = plsc.load_gather(blk_v, [sv, row_lo, li])
        hi = plsc.load_gather(blk_v, [sv, row_hi, li])
        plsc.store_scatter(out_v, [row_lo, col], lo)
        plsc.store_scatter(out_v, [row_hi, col], hi)

        @pl.when(k + NBUF < PW)
        def _():
            fire(k + NBUF, slot)

        return carry

    lax.fori_loop(0, PW, body, 0)
    pltpu.sync_copy(out_v, out_hbm.at[:, pl.ds(base, PW)])


def kernel(indices, table):
    tab_t = jnp.swapaxes(table, 0, 1)
    idx = indices.reshape(BATCH).astype(jnp.int32)
    out_t = _gather_sc(idx, tab_t)
    return jnp.swapaxes(out_t, 0, 1).reshape(1, BATCH, EMBED)


# speedup vs baseline: 1.0028x; 1.0028x over previous
import functools

import jax
import jax.numpy as jnp
from jax import lax
from jax.experimental import pallas as pl
from jax.experimental.pallas import tpu as pltpu
from jax.experimental.pallas import tpu_sc as plsc

VOCAB = 1000000
EMBED = 32
BATCH = 16384
NC, NS = 2, 16
NW = NC * NS
PW = BATCH // NW
NBUF = 8


@functools.partial(
    pl.kernel,
    mesh=plsc.VectorSubcoreMesh(core_axis_name="c", subcore_axis_name="s"),
    out_type=jax.ShapeDtypeStruct((EMBED, BATCH), jnp.float32),
    scratch_types=[
        pltpu.VMEM((PW + 16,), jnp.int32),
        pltpu.VMEM((NBUF, 4, 8, 128), jnp.float32),
        pltpu.VMEM((EMBED, PW), jnp.float32),
        pltpu.SemaphoreType.DMA,
    ],
    compiler_params=pltpu.CompilerParams(
        use_tc_tiling_on_sc=True, needs_layout_passes=False),
)
def _gather_sc(idx_hbm, tab_hbm, out_hbm, idx_v, blk_v, out_v, sem):
    wid = lax.axis_index("s") * NC + lax.axis_index("c")
    base = wid * PW
    pltpu.sync_copy(idx_hbm.at[pl.ds(base, PW)], idx_v.at[pl.ds(0, PW)])

    row_lo = lax.iota(jnp.int32, 16)
    row_hi = row_lo + 16
    tr_lo, sj_lo = row_lo >> 3, row_lo & 7
    tr_hi, sj_hi = row_hi >> 3, row_hi & 7

    def idx_at(k):
        return idx_v[pl.ds(k, 16)][0]

    def fire(k, slot):
        start = pl.multiple_of((idx_at(k) >> 7) * 128, 128)
        for tr in range(4):
            pltpu.async_copy(
                tab_hbm.at[tr, :, pl.ds(start, 128)],
                blk_v.at[slot, tr], sem)

    for b in range(NBUF):
        fire(b, b)

    def body(k, carry):
        slot = lax.rem(k, NBUF)
        for tr in range(4):
            pltpu.make_async_copy(
                tab_hbm.at[0, :, pl.ds(0, 128)],
                blk_v.at[slot, tr], sem).wait()
        li = jnp.broadcast_to(idx_at(k) & 127, (16,))
        sv = jnp.broadcast_to(slot, (16,))
        col = jnp.broadcast_to(k, (16,))
        lo = plsc.load_gather(blk_v, [sv, tr_lo, sj_lo, li])
        hi = plsc.load_gather(blk_v, [sv, tr_hi, sj_hi, li])
        plsc.store_scatter(out_v, [row_lo, col], lo)
        plsc.store_scatter(out_v, [row_hi, col], hi)

        @pl.when(k + NBUF < PW)
        def _():
            fire(k + NBUF, slot)

        return carry

    lax.fori_loop(0, PW, body, 0)
    pltpu.sync_copy(out_v, out_hbm.at[:, pl.ds(base, PW)])


def kernel(indices, table):
    tab4 = jnp.swapaxes(table, 0, 1).reshape(4, 8, VOCAB)
    idx = indices.reshape(BATCH).astype(jnp.int32)
    out_t = _gather_sc(idx, tab4)
    return jnp.swapaxes(out_t, 0, 1).reshape(1, BATCH, EMBED)


# speedup vs baseline: 1.0094x; 1.0066x over previous
import functools

import jax
import jax.numpy as jnp
from jax import lax
from jax.experimental import pallas as pl
from jax.experimental.pallas import tpu as pltpu
from jax.experimental.pallas import tpu_sc as plsc

EMBED = 32
BATCH = 16384
NC, NS = 2, 16
NW = NC * NS
PW = BATCH // NW
NBUF = 8


@functools.partial(
    pl.kernel,
    mesh=plsc.VectorSubcoreMesh(core_axis_name="c", subcore_axis_name="s"),
    out_type=jax.ShapeDtypeStruct((EMBED, BATCH), jnp.float32),
    scratch_types=[
        pltpu.VMEM((PW + 16,), jnp.int32),
        pltpu.VMEM((NBUF, EMBED, 128), jnp.float32),
        pltpu.VMEM((EMBED, PW), jnp.float32),
        pltpu.SemaphoreType.DMA,
    ],
    compiler_params=pltpu.CompilerParams(
        use_tc_tiling_on_sc=True, needs_layout_passes=False),
)
def _gather_sc(idx_hbm, tab_hbm, out_hbm, idx_v, blk_v, out_v, sem):
    wid = lax.axis_index("s") * NC + lax.axis_index("c")
    base = wid * PW
    pltpu.sync_copy(idx_hbm.at[pl.ds(base, PW)], idx_v.at[pl.ds(0, PW)])

    row_lo = lax.iota(jnp.int32, 16)
    row_hi = row_lo + 16

    def idx_at(k):
        return idx_v[pl.ds(k, 16)][0]

    def fire(k, slot):
        start = pl.multiple_of((idx_at(k) >> 7) * 128, 128)
        pltpu.async_copy(
            tab_hbm.at[:, pl.ds(start, 128)], blk_v.at[slot], sem)

    for b in range(NBUF):
        fire(b, b)

    def body(k, carry):
        slot = lax.rem(k, NBUF)
        pltpu.make_async_copy(
            tab_hbm.at[:, pl.ds(0, 128)], blk_v.at[slot], sem).wait()
        li = jnp.broadcast_to(idx_at(k) & 127, (16,))
        sv = jnp.broadcast_to(slot, (16,))
        col = jnp.broadcast_to(k, (16,))
        lo = plsc.load_gather(blk_v, [sv, row_lo, li])
        hi = plsc.load_gather(blk_v, [sv, row_hi, li])
        plsc.store_scatter(out_v, [row_lo, col], lo)
        plsc.store_scatter(out_v, [row_hi, col], hi)

        @pl.when(k + NBUF < PW)
        def _():
            fire(k + NBUF, slot)

        return carry

    lax.fori_loop(0, PW, body, 0)
    pltpu.sync_copy(out_v, out_hbm.at[:, pl.ds(base, PW)])


def kernel(indices, table):
    tab_t = jnp.swapaxes(table, 0, 1)
    idx = indices.reshape(BATCH).astype(jnp.int32)
    out_t = _gather_sc(idx, tab_t)
    return jnp.swapaxes(out_t, 0, 1).reshape(1, BATCH, EMBED)
